# Initial kernel scaffold; baseline (speedup 1.0000x reference)
import jax
import jax.numpy as jnp
from jax.experimental import pallas as pl


def kernel(x, edge_index, batch, g0_W1, g0_b1, g0_g1, g0_be1, g0_W2, g0_b2, g0_g2, g0_be2, gat_Wl, gat_bl, gat_Wr, gat_br, gat_att, gat_b, g1_W1, g1_b1, g1_g1, g1_be1, g1_W2, g1_b2, g1_g2, g1_be2, lin_W, lin_b):
    raise NotImplementedError("write your pallas kernel here")


# speedup vs baseline: 23.2627x; 23.2627x over previous
import jax
import jax.numpy as jnp
from jax import lax
from jax.experimental import pallas as pl
from jax.experimental.pallas import tpu as pltpu
from jax.experimental.pallas import tpu_sc as plsc

N = 10000
E = 320000
D = 128
B = 256
O = 64
NT = 40
NPAD = NT * 256
NTILES = 32
EW = E // NTILES
ECH = 80
ENIT = EW // ECH
RSUB = NPAD // 16


def _sc_mesh():
    return plsc.VectorSubcoreMesh(core_axis_name="c", subcore_axis_name="s")


def _seg_rows_body(x_hbm, src_hbm, dst_hbm, zrows_hbm, out_hbm,
                   srcb, dstb, rows, rows2, acc, sem, sem2):
    cid = lax.axis_index("c")
    sid = lax.axis_index("s")
    pltpu.sync_copy(zrows_hbm, acc.at[pl.ds(sid * RSUB, RSUB)])
    plsc.subcore_barrier()
    base = (cid * 16 + sid) * EW

    def body(it, _):
        off = base + it * ECH
        pltpu.sync_copy(src_hbm.at[pl.ds(off, ECH)], srcb)
        pltpu.sync_copy(dst_hbm.at[pl.ds(off, ECH)], dstb)
        pltpu.async_copy(x_hbm.at[srcb], rows, sem).wait()
        pltpu.sync_copy(rows, acc.at[dstb], add=True)
        return 0

    lax.fori_loop(0, ENIT, body, 0)
    plsc.subcore_barrier()
    pltpu.sync_copy(acc.at[pl.ds(sid * RSUB, RSUB)],
                    out_hbm.at[cid, pl.ds(sid * RSUB, RSUB)])


def _seg_rows(x_table, src, dst, zrows):
    return pl.kernel(
        _seg_rows_body,
        out_type=jax.ShapeDtypeStruct((2, NPAD, D), jnp.float32),
        mesh=_sc_mesh(),
        scratch_types=[
            pltpu.VMEM((ECH,), jnp.int32),
            pltpu.VMEM((ECH,), jnp.int32),
            pltpu.VMEM((ECH, D), jnp.float32),
            pltpu.VMEM((ECH, D), jnp.float32),
            pltpu.VMEM_SHARED((NPAD, D), jnp.float32),
            pltpu.SemaphoreType.DMA,
            pltpu.SemaphoreType.DMA,
        ],
    )(x_table, src, dst, zrows)


def _gat_edges_body(xl_hbm, xr_hbm, src_hbm, dst_hbm, attv_hbm, zcol_hbm,
                    out_hbm, xlb, xrb, srcb, dstb, attb, zacc, nacc):
    cid = lax.axis_index("c")
    sid = lax.axis_index("s")
    wid = cid * 16 + sid
    pltpu.sync_copy(zcol_hbm, zacc)
    pltpu.sync_copy(zcol_hbm, nacc)
    pltpu.sync_copy(xl_hbm, xlb)
    pltpu.sync_copy(xr_hbm, xrb)
    pltpu.sync_copy(attv_hbm, attb)
    attv = attb[...]
    base = wid * EW

    def body(it, _):
        off = base + it * ECH
        pltpu.sync_copy(src_hbm.at[pl.ds(off, ECH)], srcb)
        pltpu.sync_copy(dst_hbm.at[pl.ds(off, ECH)], dstb)
        for g in range(ECH // 16):
            s16 = srcb[pl.ds(g * 16, 16)]
            d16 = dstb[pl.ds(g * 16, 16)]
            xls = plsc.load_gather(xlb, [s16])
            xrd = plsc.load_gather(xrb, [d16])
            t = xls + xrd
            t = jnp.where(t >= 0.0, t, 0.2 * t) * attv
            w = jnp.exp(t)
            plsc.addupdate_scatter(zacc, [d16], w)
            plsc.addupdate_scatter(nacc, [d16], w * xls)
        return 0

    lax.fori_loop(0, ENIT, body, 0)
    pltpu.sync_copy(zacc, out_hbm.at[0, wid])
    pltpu.sync_copy(nacc, out_hbm.at[1, wid])


def _gat_edges(xl, xr, src, dst, attv, zcol):
    return pl.kernel(
        _gat_edges_body,
        out_type=jax.ShapeDtypeStruct((2, NTILES, NPAD), jnp.float32),
        mesh=_sc_mesh(),
        compiler_params=pltpu.CompilerParams(needs_layout_passes=False),
        scratch_types=[
            pltpu.VMEM((NPAD,), jnp.float32),
            pltpu.VMEM((NPAD,), jnp.float32),
            pltpu.VMEM((ECH,), jnp.int32),
            pltpu.VMEM((ECH,), jnp.int32),
            pltpu.VMEM((16,), jnp.float32),
            pltpu.VMEM((NPAD,), jnp.float32),
            pltpu.VMEM((NPAD,), jnp.float32),
        ],
    )(xl, xr, src, dst, attv, zcol)


def _gin0_body(x_ref, agg_ref, w1_ref, b1_ref, g1_ref, be1_ref,
               w2_ref, b2_ref, g2_ref, be2_ref, wl_ref, bl_ref, wr_ref, br_ref,
               x1_ref, xl_ref, xr_ref):
    x0 = x_ref[...] + agg_ref[0, 0:N, :] + agg_ref[1, 0:N, :]
    t = jnp.dot(x0, w1_ref[...], preferred_element_type=jnp.float32) + b1_ref[...]
    m = jnp.sum(t, axis=0, keepdims=True) * (1.0 / N)
    v = jnp.sum((t - m) ** 2, axis=0, keepdims=True) * (1.0 / N)
    h = jax.nn.relu(g1_ref[...] * (t - m) / jnp.sqrt(v + 1e-5) + be1_ref[...])
    t2 = jnp.dot(h, w2_ref[...], preferred_element_type=jnp.float32) + b2_ref[...]
    m2 = jnp.sum(t2, axis=0, keepdims=True) * (1.0 / N)
    v2 = jnp.sum((t2 - m2) ** 2, axis=0, keepdims=True) * (1.0 / N)
    x1 = jax.nn.relu(g2_ref[...] * (t2 - m2) / jnp.sqrt(v2 + 1e-5) + be2_ref[...])
    x1_ref[0:N, :] = x1
    x1_ref[N:NPAD, :] = jnp.zeros((NPAD - N, D), jnp.float32)
    xl_ref[0:N, :] = jnp.dot(x1, wl_ref[...], preferred_element_type=jnp.float32) + bl_ref[...]
    xl_ref[N:NPAD, :] = jnp.zeros((NPAD - N, 1), jnp.float32)
    xr_ref[0:N, :] = jnp.dot(x1, wr_ref[...], preferred_element_type=jnp.float32) + br_ref[...]
    xr_ref[N:NPAD, :] = jnp.zeros((NPAD - N, 1), jnp.float32)


def _attn_body(zn_ref, xlt_ref, xrt_ref, att_ref, bg_ref, attn_ref):
    z = jnp.sum(zn_ref[0], axis=0, keepdims=True)
    num = jnp.sum(zn_ref[1], axis=0, keepdims=True)
    xlt = xlt_ref[...]
    ts = xlt + xrt_ref[...]
    ws = jnp.exp(jnp.where(ts >= 0.0, ts, 0.2 * ts) * att_ref[...])
    attn_ref[...] = (num + ws * xlt) / (z + ws + 1e-16) + bg_ref[...]


def _topk_body(ac_ref, ar_ref, bc_ref, br_ref, x1_ref, jlo_ref, jhi_ref,
               xp_ref, keep_ref, cnt2_ref, n2f_ref):
    gcol = lax.broadcasted_iota(jnp.int32, (B, 1), 0)

    def cnt_body(j, acc):
        bj = br_ref[0:1, pl.ds(j * 256, 256)]
        return acc + jnp.sum((gcol == bj).astype(jnp.float32), axis=1,
                             keepdims=True)

    cntg = lax.fori_loop(0, NT, cnt_body, jnp.zeros((B, 1), jnp.float32))
    kcol = jnp.ceil(0.5 * cntg)

    cnt2 = jnp.zeros((B, 1), jnp.float32)
    for i in range(NT):
        bi = bc_ref[pl.ds(i * 256, 256), :]
        ai = ac_ref[pl.ds(i * 256, 256), :]
        gi = lax.broadcasted_iota(jnp.int32, (256, 1), 0) + i * 256

        def pair_body(j, acc):
            bj = br_ref[0:1, pl.ds(j * 256, 256)]
            aj = ar_ref[0:1, pl.ds(j * 256, 256)]
            gj = lax.broadcasted_iota(jnp.int32, (1, 256), 1) + j * 256
            gt = (aj > ai) | ((aj == ai) & (gj < gi))
            msk = ((bj == bi) & gt).astype(jnp.float32)
            return acc + jnp.sum(msk, axis=1, keepdims=True)

        rank = lax.fori_loop(jlo_ref[i], jhi_ref[i] + 1, pair_body,
                             jnp.zeros((256, 1), jnp.float32))
        oh = (bi == jnp.transpose(gcol)).astype(jnp.float32)
        kv = jnp.dot(oh, kcol, preferred_element_type=jnp.float32)
        keep = (rank < kv).astype(jnp.float32)
        keep_ref[pl.ds(i * 256, 256), :] = keep
        scale = keep * jnp.maximum(ai, 0.0)
        xp_ref[pl.ds(i * 256, 256), :] = x1_ref[pl.ds(i * 256, 256), :] * scale
        cnt2 = cnt2 + jnp.dot(jnp.transpose(oh), keep,
                              preferred_element_type=jnp.float32)
    cnt2_ref[...] = cnt2
    n2f_ref[...] = jnp.sum(cnt2, axis=0, keepdims=True)


def _final_body(xp_ref, agg_ref, keep_ref, br_ref, cnt2_ref, n2f_ref,
                w1_ref, b1_ref, g1_ref, be1_ref, w2_ref, b2_ref, g2_ref, be2_ref,
                lw_ref, lb_ref, out_ref, x2_ref):
    u = xp_ref[...] + agg_ref[0] + agg_ref[1]
    kc = keep_ref[...]
    inv = 1.0 / n2f_ref[...]
    t = jnp.dot(u, w1_ref[...], preferred_element_type=jnp.float32) + b1_ref[...]
    m = jnp.sum(t * kc, axis=0, keepdims=True) * inv
    v = jnp.sum(((t - m) ** 2) * kc, axis=0, keepdims=True) * inv
    h = jax.nn.relu(g1_ref[...] * (t - m) / jnp.sqrt(v + 1e-5) + be1_ref[...])
    t2 = jnp.dot(h, w2_ref[...], preferred_element_type=jnp.float32) + b2_ref[...]
    m2 = jnp.sum(t2 * kc, axis=0, keepdims=True) * inv
    v2 = jnp.sum(((t2 - m2) ** 2) * kc, axis=0, keepdims=True) * inv
    x2 = jax.nn.relu(g2_ref[...] * (t2 - m2) / jnp.sqrt(v2 + 1e-5) + be2_ref[...])
    x2_ref[...] = x2 * kc
    gcol = lax.broadcasted_iota(jnp.int32, (B, 1), 0)

    def pool_body(j, acc):
        bj = br_ref[0:1, pl.ds(j * 256, 256)]
        oh = (gcol == bj).astype(jnp.float32)
        return acc + jnp.dot(oh, x2_ref[pl.ds(j * 256, 256), :],
                             preferred_element_type=jnp.float32)

    sums = lax.fori_loop(0, NT, pool_body, jnp.zeros((B, D), jnp.float32))
    pooled = sums / jnp.maximum(cnt2_ref[...], 1.0)
    out_ref[...] = jnp.dot(pooled, lw_ref[...],
                           preferred_element_type=jnp.float32) + lb_ref[...]


def kernel(x, edge_index, batch, g0_W1, g0_b1, g0_g1, g0_be1, g0_W2, g0_b2,
           g0_g2, g0_be2, gat_Wl, gat_bl, gat_Wr, gat_br, gat_att, gat_b,
           g1_W1, g1_b1, g1_g1, g1_be1, g1_W2, g1_b2, g1_g2, g1_be2,
           lin_W, lin_b):
    f32 = jnp.float32
    src = edge_index[0]
    dst = edge_index[1]
    zrows = jnp.zeros((RSUB, D), f32)
    zcol = jnp.zeros((NPAD,), f32)

    agg0 = _seg_rows(x, src, dst, zrows)
    x1, xl, xr = pl.pallas_call(
        _gin0_body,
        out_shape=[
            jax.ShapeDtypeStruct((NPAD, D), f32),
            jax.ShapeDtypeStruct((NPAD, 1), f32),
            jax.ShapeDtypeStruct((NPAD, 1), f32),
        ],
    )(x, agg0, g0_W1.T, g0_b1.reshape(1, D), g0_g1.reshape(1, D),
      g0_be1.reshape(1, D), g0_W2.T, g0_b2.reshape(1, D), g0_g2.reshape(1, D),
      g0_be2.reshape(1, D), gat_Wl.T, gat_bl.reshape(1, 1), gat_Wr.T,
      gat_br.reshape(1, 1))

    attv = jnp.broadcast_to(gat_att, (16,))
    zn = _gat_edges(xl[:, 0], xr[:, 0], src, dst, attv, zcol)
    attn_row = pl.pallas_call(
        _attn_body,
        out_shape=jax.ShapeDtypeStruct((1, NPAD), f32),
    )(zn, xl.reshape(1, NPAD), xr.reshape(1, NPAD), gat_att.reshape(1, 1),
      gat_b.reshape(1, 1))

    attn_col = attn_row.reshape(NPAD, 1)
    batch_pad = jnp.pad(batch, (0, NPAD - N), constant_values=1 << 20)
    batch_col = batch_pad.reshape(NPAD, 1)
    batch_row = batch_pad.reshape(1, NPAD)
    tlo = batch_pad[::256]
    thi = batch_pad[255::256]
    jlo = jnp.searchsorted(thi, tlo, side="left").astype(jnp.int32)
    jhi = (jnp.searchsorted(tlo, thi, side="right") - 1).astype(jnp.int32)
    xp, keep, cnt2, n2f = pl.pallas_call(
        _topk_body,
        in_specs=[
            pl.BlockSpec(memory_space=pltpu.VMEM),
            pl.BlockSpec(memory_space=pltpu.VMEM),
            pl.BlockSpec(memory_space=pltpu.VMEM),
            pl.BlockSpec(memory_space=pltpu.VMEM),
            pl.BlockSpec(memory_space=pltpu.VMEM),
            pl.BlockSpec(memory_space=pltpu.SMEM),
            pl.BlockSpec(memory_space=pltpu.SMEM),
        ],
        out_shape=[
            jax.ShapeDtypeStruct((NPAD, D), f32),
            jax.ShapeDtypeStruct((NPAD, 1), f32),
            jax.ShapeDtypeStruct((B, 1), f32),
            jax.ShapeDtypeStruct((1, 1), f32),
        ],
    )(attn_col, attn_row, batch_col, batch_row, x1, jlo, jhi)

    agg1 = _seg_rows(xp, src, dst, zrows)
    out = pl.pallas_call(
        _final_body,
        out_shape=jax.ShapeDtypeStruct((B, O), f32),
        scratch_shapes=[pltpu.VMEM((NPAD, D), f32)],
    )(xp, agg1, keep, batch_row, cnt2, n2f, g1_W1.T, g1_b1.reshape(1, D),
      g1_g1.reshape(1, D), g1_be1.reshape(1, D), g1_W2.T, g1_b2.reshape(1, D),
      g1_g2.reshape(1, D), g1_be2.reshape(1, D), lin_W.T, lin_b.reshape(1, O))
    return out


# speedup vs baseline: 42.2669x; 1.8169x over previous
import jax
import jax.numpy as jnp
from jax import lax
from jax.experimental import pallas as pl
from jax.experimental.pallas import tpu as pltpu
from jax.experimental.pallas import tpu_sc as plsc

N = 10000
E = 320000
D = 128
B = 256
O = 64
NT = 40
NPAD = NT * 256
NTILES = 32
EW = E // NTILES
ECH = 80
ENIT = EW // ECH
RSUB = NPAD // 16


def _sc_mesh():
    return plsc.VectorSubcoreMesh(core_axis_name="c", subcore_axis_name="s")


def _seg_rows_body(x_hbm, src3_hbm, dst3_hbm, zrows_hbm, out_hbm,
                   srcb0, srcb1, dstball, rows, rows2, acc, sem, sem2):
    cid = lax.axis_index("c")
    sid = lax.axis_index("s")
    wid = cid * 16 + sid
    pltpu.sync_copy(zrows_hbm, acc.at[pl.ds(sid * RSUB, RSUB)])
    pltpu.sync_copy(dst3_hbm.at[wid], dstball)
    plsc.subcore_barrier()
    pltpu.sync_copy(src3_hbm.at[wid, 0], srcb0)
    pltpu.async_copy(x_hbm.at[srcb0], rows, sem)

    def body(it, _):
        @pl.when(it % 2 == 0)
        def _even():
            @pl.when(it + 1 < ENIT)
            def _():
                pltpu.sync_copy(src3_hbm.at[wid, it + 1], srcb1)
            pltpu.make_async_copy(x_hbm.at[srcb0], rows, sem).wait()

            @pl.when(it + 1 < ENIT)
            def _():
                pltpu.async_copy(x_hbm.at[srcb1], rows2, sem2)
            pltpu.sync_copy(rows, acc.at[dstball.at[it]], add=True)

        @pl.when(it % 2 == 1)
        def _odd():
            @pl.when(it + 1 < ENIT)
            def _():
                pltpu.sync_copy(src3_hbm.at[wid, it + 1], srcb0)
            pltpu.make_async_copy(x_hbm.at[srcb1], rows2, sem2).wait()

            @pl.when(it + 1 < ENIT)
            def _():
                pltpu.async_copy(x_hbm.at[srcb0], rows, sem)
            pltpu.sync_copy(rows2, acc.at[dstball.at[it]], add=True)
        return 0

    lax.fori_loop(0, ENIT, body, 0)
    plsc.subcore_barrier()
    pltpu.sync_copy(acc.at[pl.ds(sid * RSUB, RSUB)],
                    out_hbm.at[cid, pl.ds(sid * RSUB, RSUB)])


def _seg_rows(x_table, src3, dst3, zrows):
    return pl.kernel(
        _seg_rows_body,
        out_type=jax.ShapeDtypeStruct((2, NPAD, D), jnp.float32),
        mesh=_sc_mesh(),
        scratch_types=[
            pltpu.VMEM((ECH,), jnp.int32),
            pltpu.VMEM((ECH,), jnp.int32),
            pltpu.VMEM((ENIT, ECH), jnp.int32),
            pltpu.VMEM((ECH, D), jnp.float32),
            pltpu.VMEM((ECH, D), jnp.float32),
            pltpu.VMEM_SHARED((NPAD, D), jnp.float32),
            pltpu.SemaphoreType.DMA,
            pltpu.SemaphoreType.DMA,
        ],
    )(x_table, src3, dst3, zrows)


def _gat_edges_body(xl_hbm, xr_hbm, src3_hbm, dst3_hbm, attv_hbm, zcol_hbm,
                    out_hbm, xlb, xrb, srcball, dstball, attb, zacc, nacc):
    cid = lax.axis_index("c")
    sid = lax.axis_index("s")
    wid = cid * 16 + sid
    pltpu.sync_copy(zcol_hbm, zacc)
    pltpu.sync_copy(zcol_hbm, nacc)
    pltpu.sync_copy(xl_hbm, xlb)
    pltpu.sync_copy(xr_hbm, xrb)
    pltpu.sync_copy(attv_hbm, attb)
    pltpu.sync_copy(src3_hbm.at[wid], srcball)
    pltpu.sync_copy(dst3_hbm.at[wid], dstball)
    attv = attb[...]

    def body(it, _):
        for g in range(ECH // 16):
            s16 = srcball[it, pl.ds(g * 16, 16)]
            d16 = dstball[it, pl.ds(g * 16, 16)]
            xls = plsc.load_gather(xlb, [s16])
            xrd = plsc.load_gather(xrb, [d16])
            t = xls + xrd
            t = jnp.where(t >= 0.0, t, 0.2 * t) * attv
            w = jnp.exp(t)
            plsc.addupdate_scatter(zacc, [d16], w)
            plsc.addupdate_scatter(nacc, [d16], w * xls)
        return 0

    lax.fori_loop(0, ENIT, body, 0)
    pltpu.sync_copy(zacc, out_hbm.at[0, wid])
    pltpu.sync_copy(nacc, out_hbm.at[1, wid])


def _gat_edges(xl, xr, src3, dst3, attv, zcol):
    return pl.kernel(
        _gat_edges_body,
        out_type=jax.ShapeDtypeStruct((2, NTILES, NPAD), jnp.float32),
        mesh=_sc_mesh(),
        compiler_params=pltpu.CompilerParams(needs_layout_passes=False),
        scratch_types=[
            pltpu.VMEM((NPAD,), jnp.float32),
            pltpu.VMEM((NPAD,), jnp.float32),
            pltpu.VMEM((ENIT, ECH), jnp.int32),
            pltpu.VMEM((ENIT, ECH), jnp.int32),
            pltpu.VMEM((16,), jnp.float32),
            pltpu.VMEM((NPAD,), jnp.float32),
            pltpu.VMEM((NPAD,), jnp.float32),
        ],
    )(xl, xr, src3, dst3, attv, zcol)


def _gin0_body(x_ref, agg_ref, w1_ref, b1_ref, g1_ref, be1_ref,
               w2_ref, b2_ref, g2_ref, be2_ref, wl_ref, bl_ref, wr_ref, br_ref,
               x1_ref, xl_ref, xr_ref):
    x0 = x_ref[...] + agg_ref[0, 0:N, :] + agg_ref[1, 0:N, :]
    t = jnp.dot(x0, w1_ref[...], preferred_element_type=jnp.float32) + b1_ref[...]
    m = jnp.sum(t, axis=0, keepdims=True) * (1.0 / N)
    v = jnp.sum((t - m) ** 2, axis=0, keepdims=True) * (1.0 / N)
    h = jax.nn.relu(g1_ref[...] * (t - m) / jnp.sqrt(v + 1e-5) + be1_ref[...])
    t2 = jnp.dot(h, w2_ref[...], preferred_element_type=jnp.float32) + b2_ref[...]
    m2 = jnp.sum(t2, axis=0, keepdims=True) * (1.0 / N)
    v2 = jnp.sum((t2 - m2) ** 2, axis=0, keepdims=True) * (1.0 / N)
    x1 = jax.nn.relu(g2_ref[...] * (t2 - m2) / jnp.sqrt(v2 + 1e-5) + be2_ref[...])
    x1_ref[0:N, :] = x1
    x1_ref[N:NPAD, :] = jnp.zeros((NPAD - N, D), jnp.float32)
    xl_ref[0:N, :] = jnp.dot(x1, wl_ref[...], preferred_element_type=jnp.float32) + bl_ref[...]
    xl_ref[N:NPAD, :] = jnp.zeros((NPAD - N, 1), jnp.float32)
    xr_ref[0:N, :] = jnp.dot(x1, wr_ref[...], preferred_element_type=jnp.float32) + br_ref[...]
    xr_ref[N:NPAD, :] = jnp.zeros((NPAD - N, 1), jnp.float32)


def _attn_body(zn_ref, xlt_ref, xrt_ref, att_ref, bg_ref, attn_ref):
    z = jnp.sum(zn_ref[0], axis=0, keepdims=True)
    num = jnp.sum(zn_ref[1], axis=0, keepdims=True)
    xlt = xlt_ref[...]
    ts = xlt + xrt_ref[...]
    ws = jnp.exp(jnp.where(ts >= 0.0, ts, 0.2 * ts) * att_ref[...])
    attn_ref[...] = (num + ws * xlt) / (z + ws + 1e-16) + bg_ref[...]


def _topk_body(ac_ref, ar_ref, bc_ref, br_ref, x1_ref, jlo_ref, jhi_ref,
               xp_ref, keep_ref, cnt2_ref, n2f_ref):
    gcol = lax.broadcasted_iota(jnp.int32, (B, 1), 0)

    def cnt_body(j, acc):
        bj = br_ref[0:1, pl.ds(j * 256, 256)]
        return acc + jnp.sum((gcol == bj).astype(jnp.float32), axis=1,
                             keepdims=True)

    cntg = lax.fori_loop(0, NT, cnt_body, jnp.zeros((B, 1), jnp.float32))
    kcol = jnp.ceil(0.5 * cntg)

    cnt2 = jnp.zeros((B, 1), jnp.float32)
    for i in range(NT):
        bi = bc_ref[pl.ds(i * 256, 256), :]
        ai = ac_ref[pl.ds(i * 256, 256), :]
        gi = lax.broadcasted_iota(jnp.int32, (256, 1), 0) + i * 256

        def pair_body(j, acc):
            bj = br_ref[0:1, pl.ds(j * 256, 256)]
            aj = ar_ref[0:1, pl.ds(j * 256, 256)]
            gj = lax.broadcasted_iota(jnp.int32, (1, 256), 1) + j * 256
            gt = (aj > ai) | ((aj == ai) & (gj < gi))
            msk = ((bj == bi) & gt).astype(jnp.float32)
            return acc + jnp.sum(msk, axis=1, keepdims=True)

        rank = lax.fori_loop(jlo_ref[i], jhi_ref[i] + 1, pair_body,
                             jnp.zeros((256, 1), jnp.float32))
        oh = (bi == jnp.transpose(gcol)).astype(jnp.float32)
        kv = jnp.dot(oh, kcol, preferred_element_type=jnp.float32)
        keep = (rank < kv).astype(jnp.float32)
        keep_ref[pl.ds(i * 256, 256), :] = keep
        scale = keep * jnp.maximum(ai, 0.0)
        xp_ref[pl.ds(i * 256, 256), :] = x1_ref[pl.ds(i * 256, 256), :] * scale
        cnt2 = cnt2 + jnp.dot(jnp.transpose(oh), keep,
                              preferred_element_type=jnp.float32)
    cnt2_ref[...] = cnt2
    n2f_ref[...] = jnp.sum(cnt2, axis=0, keepdims=True)


def _final_body(xp_ref, agg_ref, keep_ref, br_ref, cnt2_ref, n2f_ref,
                w1_ref, b1_ref, g1_ref, be1_ref, w2_ref, b2_ref, g2_ref, be2_ref,
                lw_ref, lb_ref, out_ref, x2_ref):
    u = xp_ref[...] + agg_ref[0] + agg_ref[1]
    kc = keep_ref[...]
    inv = 1.0 / n2f_ref[...]
    t = jnp.dot(u, w1_ref[...], preferred_element_type=jnp.float32) + b1_ref[...]
    m = jnp.sum(t * kc, axis=0, keepdims=True) * inv
    v = jnp.sum(((t - m) ** 2) * kc, axis=0, keepdims=True) * inv
    h = jax.nn.relu(g1_ref[...] * (t - m) / jnp.sqrt(v + 1e-5) + be1_ref[...])
    t2 = jnp.dot(h, w2_ref[...], preferred_element_type=jnp.float32) + b2_ref[...]
    m2 = jnp.sum(t2 * kc, axis=0, keepdims=True) * inv
    v2 = jnp.sum(((t2 - m2) ** 2) * kc, axis=0, keepdims=True) * inv
    x2 = jax.nn.relu(g2_ref[...] * (t2 - m2) / jnp.sqrt(v2 + 1e-5) + be2_ref[...])
    x2_ref[...] = x2 * kc
    gcol = lax.broadcasted_iota(jnp.int32, (B, 1), 0)

    def pool_body(j, acc):
        bj = br_ref[0:1, pl.ds(j * 256, 256)]
        oh = (gcol == bj).astype(jnp.float32)
        return acc + jnp.dot(oh, x2_ref[pl.ds(j * 256, 256), :],
                             preferred_element_type=jnp.float32)

    sums = lax.fori_loop(0, NT, pool_body, jnp.zeros((B, D), jnp.float32))
    pooled = sums / jnp.maximum(cnt2_ref[...], 1.0)
    out_ref[...] = jnp.dot(pooled, lw_ref[...],
                           preferred_element_type=jnp.float32) + lb_ref[...]


def kernel(x, edge_index, batch, g0_W1, g0_b1, g0_g1, g0_be1, g0_W2, g0_b2,
           g0_g2, g0_be2, gat_Wl, gat_bl, gat_Wr, gat_br, gat_att, gat_b,
           g1_W1, g1_b1, g1_g1, g1_be1, g1_W2, g1_b2, g1_g2, g1_be2,
           lin_W, lin_b):
    f32 = jnp.float32
    src3 = edge_index[0].reshape(NTILES, ENIT, ECH)
    dst3 = edge_index[1].reshape(NTILES, ENIT, ECH)
    zrows = jnp.zeros((RSUB, D), f32)
    zcol = jnp.zeros((NPAD,), f32)

    agg0 = _seg_rows(x, src3, dst3, zrows)
    x1, xl, xr = pl.pallas_call(
        _gin0_body,
        out_shape=[
            jax.ShapeDtypeStruct((NPAD, D), f32),
            jax.ShapeDtypeStruct((NPAD, 1), f32),
            jax.ShapeDtypeStruct((NPAD, 1), f32),
        ],
    )(x, agg0, g0_W1.T, g0_b1.reshape(1, D), g0_g1.reshape(1, D),
      g0_be1.reshape(1, D), g0_W2.T, g0_b2.reshape(1, D), g0_g2.reshape(1, D),
      g0_be2.reshape(1, D), gat_Wl.T, gat_bl.reshape(1, 1), gat_Wr.T,
      gat_br.reshape(1, 1))

    attv = jnp.broadcast_to(gat_att, (16,))
    zn = _gat_edges(xl[:, 0], xr[:, 0], src3, dst3, attv, zcol)
    attn_row = pl.pallas_call(
        _attn_body,
        out_shape=jax.ShapeDtypeStruct((1, NPAD), f32),
    )(zn, xl.reshape(1, NPAD), xr.reshape(1, NPAD), gat_att.reshape(1, 1),
      gat_b.reshape(1, 1))

    attn_col = attn_row.reshape(NPAD, 1)
    batch_pad = jnp.pad(batch, (0, NPAD - N), constant_values=1 << 20)
    batch_col = batch_pad.reshape(NPAD, 1)
    batch_row = batch_pad.reshape(1, NPAD)
    tlo = batch_pad[::256]
    thi = batch_pad[255::256]
    jlo = jnp.searchsorted(thi, tlo, side="left").astype(jnp.int32)
    jhi = (jnp.searchsorted(tlo, thi, side="right") - 1).astype(jnp.int32)
    xp, keep, cnt2, n2f = pl.pallas_call(
        _topk_body,
        in_specs=[
            pl.BlockSpec(memory_space=pltpu.VMEM),
            pl.BlockSpec(memory_space=pltpu.VMEM),
            pl.BlockSpec(memory_space=pltpu.VMEM),
            pl.BlockSpec(memory_space=pltpu.VMEM),
            pl.BlockSpec(memory_space=pltpu.VMEM),
            pl.BlockSpec(memory_space=pltpu.SMEM),
            pl.BlockSpec(memory_space=pltpu.SMEM),
        ],
        out_shape=[
            jax.ShapeDtypeStruct((NPAD, D), f32),
            jax.ShapeDtypeStruct((NPAD, 1), f32),
            jax.ShapeDtypeStruct((B, 1), f32),
            jax.ShapeDtypeStruct((1, 1), f32),
        ],
    )(attn_col, attn_row, batch_col, batch_row, x1, jlo, jhi)

    agg1 = _seg_rows(xp, src3, dst3, zrows)
    out = pl.pallas_call(
        _final_body,
        out_shape=jax.ShapeDtypeStruct((B, O), f32),
        scratch_shapes=[pltpu.VMEM((NPAD, D), f32)],
    )(xp, agg1, keep, batch_row, cnt2, n2f, g1_W1.T, g1_b1.reshape(1, D),
      g1_g1.reshape(1, D), g1_be1.reshape(1, D), g1_W2.T, g1_b2.reshape(1, D),
      g1_g2.reshape(1, D), g1_be2.reshape(1, D), lin_W.T, lin_b.reshape(1, O))
    return out
